# Initial kernel scaffold; baseline (speedup 1.0000x reference)
import jax
import jax.numpy as jnp
from jax.experimental import pallas as pl


def kernel(x, edge_index, edge_weight, w_ih, w_hh, b_ih):
    raise NotImplementedError("write your pallas kernel here")


# speedup vs baseline: 5.4883x; 5.4883x over previous
import functools

import jax
import jax.numpy as jnp
from jax import lax
from jax.experimental import pallas as pl
from jax.experimental.pallas import tpu as pltpu
from jax.experimental.pallas import tpu_sc as plsc

T, N, F, H, E = 12, 10000, 128, 128, 320000
ALPHA = 0.9

NC, NS = 2, 16
NW = NC * NS
E_PER_W = E // NW
CHUNK = 80
NCHUNK = E_PER_W // CHUNK
N_PAD = 10240
ROWS_PER_S = N_PAD // NS
ZROWS = 128


def _sc_scatter_rows(table, gidx, sidx, ew):
    mesh = plsc.VectorSubcoreMesh(core_axis_name="c", subcore_axis_name="s")

    @functools.partial(
        pl.kernel,
        mesh=mesh,
        out_type=jax.ShapeDtypeStruct((NC, N_PAD, H), jnp.float32),
        scratch_types=[
            pltpu.VMEM((CHUNK,), jnp.int32),
            pltpu.VMEM((CHUNK,), jnp.int32),
            pltpu.VMEM((CHUNK,), jnp.float32),
            pltpu.VMEM((CHUNK, H), jnp.float32),
            pltpu.VMEM((ZROWS, H), jnp.float32),
            pltpu.VMEM_SHARED((N_PAD, H), jnp.float32),
            pltpu.SemaphoreType.DMA,
        ],
    )
    def k(table_hbm, gidx_hbm, sidx_hbm, ew_hbm, out_hbm,
          gi_v, si_v, w_v, rows_v, zb_v, acc_sh, sem):
        c = lax.axis_index("c")
        s = lax.axis_index("s")
        wid = c * NS + s
        base = wid * E_PER_W

        zero16 = jnp.zeros((16,), jnp.float32)

        def zrow(i, carry):
            for j in range(H // 16):
                zb_v[i, pl.ds(j * 16, 16)] = zero16
            return carry

        lax.fori_loop(0, ZROWS, zrow, 0)
        for k2 in range(ROWS_PER_S // ZROWS):
            pltpu.sync_copy(zb_v,
                            acc_sh.at[pl.ds(s * ROWS_PER_S + k2 * ZROWS, ZROWS)])
        plsc.subcore_barrier()

        def chunk(ci, carry):
            eo = base + ci * CHUNK
            pltpu.sync_copy(gidx_hbm.at[pl.ds(eo, CHUNK)], gi_v)
            pltpu.sync_copy(sidx_hbm.at[pl.ds(eo, CHUNK)], si_v)
            pltpu.sync_copy(ew_hbm.at[pl.ds(eo, CHUNK)], w_v)
            pltpu.async_copy(table_hbm.at[gi_v], rows_v, sem).wait()

            def scale_group(g, c2):
                wv = w_v[pl.ds(g * 16, 16)]
                for r in range(16):
                    wgt = wv[r]
                    row = g * 16 + r
                    for j in range(H // 16):
                        sl = pl.ds(j * 16, 16)
                        rows_v[row, sl] = rows_v[row, sl] * wgt
                return c2

            lax.fori_loop(0, CHUNK // 16, scale_group, 0)
            pltpu.sync_copy(rows_v, acc_sh.at[si_v], add=True)
            return carry

        lax.fori_loop(0, NCHUNK, chunk, 0)
        plsc.subcore_barrier()
        pltpu.sync_copy(acc_sh.at[pl.ds(s * ROWS_PER_S, ROWS_PER_S)],
                        out_hbm.at[c, pl.ds(s * ROWS_PER_S, ROWS_PER_S)])

    return k(table, gidx, sidx, ew)


_BR = 1000


def _tc_input_proj(x2, w_ih, b2):
    def body(xr, wr, br, outr):
        outr[...] = lax.dot_general(
            xr[...], wr[...], (((1,), (1,)), ((), ())),
            preferred_element_type=jnp.float32) + br[...]

    rows = T * N
    return pl.pallas_call(
        body,
        grid=(rows // _BR,),
        in_specs=[
            pl.BlockSpec((_BR, F), lambda i: (i, 0)),
            pl.BlockSpec((H, F), lambda i: (0, 0)),
            pl.BlockSpec((1, H), lambda i: (0, 0)),
        ],
        out_specs=pl.BlockSpec((_BR, H), lambda i: (i, 0)),
        out_shape=jax.ShapeDtypeStruct((rows, H), jnp.float32),
    )(x2, w_ih, b2)


def _tc_invdeg(d0, d1):
    def body(a, b, o):
        o[...] = 1.0 / (a[...] + b[...] + 1.0)

    return pl.pallas_call(
        body,
        grid=(N // _BR,),
        in_specs=[pl.BlockSpec((_BR, H), lambda i: (i, 0))] * 2,
        out_specs=pl.BlockSpec((_BR, H), lambda i: (i, 0)),
        out_shape=jax.ShapeDtypeStruct((N, H), jnp.float32),
    )(d0, d1)


def _tc_update(h, hd, m0, m1, ip_t, invdeg, w_hh):
    def body(hr, hdr, m0r, m1r, ipr, idr, wr, hor, hdor):
        z = m0r[...] + m1r[...] + hdr[...]
        u = lax.dot_general(z, wr[...], (((1,), (1,)), ((), ())),
                            preferred_element_type=jnp.float32)
        hn = (1.0 - ALPHA) * hr[...] + ALPHA * jnp.tanh(ipr[...] + u)
        hor[...] = hn
        hdor[...] = hn * idr[...]

    blk = pl.BlockSpec((_BR, H), lambda i: (i, 0))
    return pl.pallas_call(
        body,
        grid=(N // _BR,),
        in_specs=[blk, blk, blk, blk, blk,
                  blk, pl.BlockSpec((H, H), lambda i: (0, 0))],
        out_specs=[blk, blk],
        out_shape=[jax.ShapeDtypeStruct((N, H), jnp.float32)] * 2,
    )(h, hd, m0, m1, ip_t, invdeg, w_hh)


def kernel(x, edge_index, edge_weight, w_ih, w_hh, b_ih):
    src = edge_index[0]
    dst = edge_index[1]

    ones = jnp.ones((N, H), jnp.float32)
    degp = _sc_scatter_rows(ones, src, src, edge_weight)
    invdeg = _tc_invdeg(degp[0, :N], degp[1, :N])

    ip = _tc_input_proj(x.reshape(T * N, F), w_ih, b_ih.reshape(1, H))
    ip = ip.reshape(T, N, H)

    zero = jnp.zeros((N, H), jnp.float32)
    h, hd, m0, m1 = zero, zero, zero, zero
    outs = []
    for t in range(T):
        h, hd = _tc_update(h, hd, m0, m1, ip[t], invdeg, w_hh)
        outs.append(h)
        if t + 1 < T:
            mp = _sc_scatter_rows(hd, src, dst, edge_weight)
            m0, m1 = mp[0, :N], mp[1, :N]

    return jnp.stack(outs)[:, :, None, :]


# speedup vs baseline: 13.8292x; 2.5197x over previous
import functools

import jax
import jax.numpy as jnp
from jax import lax
from jax.experimental import pallas as pl
from jax.experimental.pallas import tpu as pltpu
from jax.experimental.pallas import tpu_sc as plsc

T, N, F, H, E = 12, 10000, 128, 128, 320000
ALPHA = 0.9

NC, NS = 2, 16
NW = NC * NS
E_PER_W = E // NW
CHUNK = 80
NCHUNK = E_PER_W // CHUNK
NBUF = 2
N_PAD = 10112
ROWS_PER_S = N_PAD // NS


def _sc_scatter_rows(table, gidx2, sidx3, ew):
    mesh = plsc.VectorSubcoreMesh(core_axis_name="c", subcore_axis_name="s")

    @functools.partial(
        pl.kernel,
        mesh=mesh,
        out_type=jax.ShapeDtypeStruct((NC, N_PAD, H), jnp.float32),
        scratch_types=[
            pltpu.VMEM((E_PER_W,), jnp.int32),
            pltpu.VMEM((NCHUNK, CHUNK), jnp.int32),
            pltpu.VMEM((NBUF, CHUNK), jnp.float32),
            pltpu.VMEM((NBUF, CHUNK, H), jnp.float32),
            pltpu.VMEM_SHARED((N_PAD, H), jnp.float32),
            pltpu.SemaphoreType.DMA,
            pltpu.SemaphoreType.DMA,
            pltpu.SemaphoreType.DMA,
            pltpu.SemaphoreType.DMA,
        ],
    )
    def k(table_hbm, gidx_hbm, sidx_hbm, ew_hbm, out_hbm,
          gi_v, si_v, ws_v, rows_v, acc_sh, *sems):
        c = lax.axis_index("c")
        s = lax.axis_index("s")
        wid = c * NS + s

        pltpu.sync_copy(gidx_hbm.at[wid], gi_v)
        pltpu.sync_copy(sidx_hbm.at[wid], si_v)

        zero16 = jnp.zeros((16,), jnp.float32)

        def zrow(i, carry):
            for j in range(H // 16):
                rows_v[0, i, pl.ds(j * 16, 16)] = zero16
            return carry

        lax.fori_loop(0, CHUNK, zrow, 0)
        zoff = s * ROWS_PER_S
        for k2 in range(ROWS_PER_S // CHUNK):
            pltpu.sync_copy(rows_v.at[0],
                            acc_sh.at[pl.ds(zoff + k2 * CHUNK, CHUNK)])
        ztail = ROWS_PER_S % CHUNK
        if ztail:
            pltpu.sync_copy(
                rows_v.at[0, pl.ds(0, ztail)],
                acc_sh.at[pl.ds(zoff + ROWS_PER_S - ztail, ztail)])
        plsc.subcore_barrier()

        def g_start(ci, b):
            pltpu.async_copy(table_hbm.at[gi_v.at[pl.ds(ci * CHUNK, CHUNK)]],
                             rows_v.at[b], sems[b])
            pltpu.async_copy(
                ew_hbm.at[pl.ds((wid * NCHUNK + ci) * CHUNK, CHUNK)],
                ws_v.at[b], sems[NBUF + b])

        def g_wait(b):
            pltpu.make_async_copy(table_hbm.at[pl.ds(0, CHUNK)],
                                  rows_v.at[b], sems[b]).wait()
            pltpu.make_async_copy(ew_hbm.at[pl.ds(0, CHUNK)],
                                  ws_v.at[b], sems[NBUF + b]).wait()

        def process(ci, b, refill):
            g_wait(b)
            for g in range(CHUNK // 16):
                wv = ws_v[b, pl.ds(g * 16, 16)]
                for r in range(16):
                    wgt = wv[r]
                    row = g * 16 + r
                    for j in range(H // 16):
                        sl = pl.ds(j * 16, 16)
                        rows_v[b, row, sl] = rows_v[b, row, sl] * wgt
            pltpu.sync_copy(rows_v.at[b], acc_sh.at[si_v.at[ci]], add=True)

            if refill:
                @pl.when(ci + NBUF < NCHUNK)
                def _():
                    g_start(ci + NBUF, b)

        for b in range(NBUF):
            g_start(b, b)

        def chunk(i, carry):
            for b in range(NBUF):
                process(i * NBUF + b, b, True)
            return carry

        lax.fori_loop(0, NCHUNK // NBUF, chunk, 0)
        for ci in range(NCHUNK - NCHUNK % NBUF, NCHUNK):
            process(ci, ci % NBUF, False)

        plsc.subcore_barrier()
        pltpu.sync_copy(acc_sh.at[pl.ds(s * ROWS_PER_S, ROWS_PER_S)],
                        out_hbm.at[c, pl.ds(s * ROWS_PER_S, ROWS_PER_S)])

    return k(table, gidx2, sidx3, ew)


_BR = 1000


def _tc_input_proj(x2, w_ih, b2):
    def body(xr, wr, br, outr):
        outr[...] = lax.dot_general(
            xr[...], wr[...], (((1,), (1,)), ((), ())),
            preferred_element_type=jnp.float32) + br[...]

    rows = T * N
    return pl.pallas_call(
        body,
        grid=(rows // _BR,),
        in_specs=[
            pl.BlockSpec((_BR, F), lambda i: (i, 0)),
            pl.BlockSpec((H, F), lambda i: (0, 0)),
            pl.BlockSpec((1, H), lambda i: (0, 0)),
        ],
        out_specs=pl.BlockSpec((_BR, H), lambda i: (i, 0)),
        out_shape=jax.ShapeDtypeStruct((rows, H), jnp.float32),
    )(x2, w_ih, b2)


def _tc_invdeg(d0, d1):
    def body(a, b, o):
        o[...] = 1.0 / (a[...] + b[...] + 1.0)

    return pl.pallas_call(
        body,
        grid=(N // _BR,),
        in_specs=[pl.BlockSpec((_BR, H), lambda i: (i, 0))] * 2,
        out_specs=pl.BlockSpec((_BR, H), lambda i: (i, 0)),
        out_shape=jax.ShapeDtypeStruct((N, H), jnp.float32),
    )(d0, d1)


def _tc_update(h, hd, m0, m1, ip_t, invdeg, w_hh):
    def body(hr, hdr, m0r, m1r, ipr, idr, wr, hor, hdor):
        z = m0r[...] + m1r[...] + hdr[...]
        u = lax.dot_general(z, wr[...], (((1,), (1,)), ((), ())),
                            preferred_element_type=jnp.float32)
        hn = (1.0 - ALPHA) * hr[...] + ALPHA * jnp.tanh(ipr[...] + u)
        hor[...] = hn
        hdor[...] = hn * idr[...]

    blk = pl.BlockSpec((_BR, H), lambda i: (i, 0))
    return pl.pallas_call(
        body,
        grid=(N // _BR,),
        in_specs=[blk, blk, blk, blk, blk,
                  blk, pl.BlockSpec((H, H), lambda i: (0, 0))],
        out_specs=[blk, blk],
        out_shape=[jax.ShapeDtypeStruct((N, H), jnp.float32)] * 2,
    )(h, hd, m0, m1, ip_t, invdeg, w_hh)


def kernel(x, edge_index, edge_weight, w_ih, w_hh, b_ih):
    src = edge_index[0]
    dst = edge_index[1]

    gidx2 = src.reshape(NW, E_PER_W)
    ssrc3 = src.reshape(NW, NCHUNK, CHUNK)
    sdst3 = dst.reshape(NW, NCHUNK, CHUNK)
    ew = edge_weight

    ones = jnp.ones((N, H), jnp.float32)
    degp = _sc_scatter_rows(ones, gidx2, ssrc3, ew)
    invdeg = _tc_invdeg(degp[0, :N], degp[1, :N])

    ip = _tc_input_proj(x.reshape(T * N, F), w_ih, b_ih.reshape(1, H))
    ip = ip.reshape(T, N, H)

    zero = jnp.zeros((N, H), jnp.float32)
    h, hd, m0, m1 = zero, zero, zero, zero
    outs = []
    for t in range(T):
        h, hd = _tc_update(h, hd, m0, m1, ip[t], invdeg, w_hh)
        outs.append(h)
        if t + 1 < T:
            mp = _sc_scatter_rows(hd, gidx2, sdst3, ew)
            m0, m1 = mp[0, :N], mp[1, :N]

    return jnp.stack(outs)[:, :, None, :]


# speedup vs baseline: 15.2050x; 1.0995x over previous
import functools

import jax
import jax.numpy as jnp
from jax import lax
from jax.experimental import pallas as pl
from jax.experimental.pallas import tpu as pltpu
from jax.experimental.pallas import tpu_sc as plsc

T, N, F, H, E = 12, 10000, 128, 128, 320000
ALPHA = 0.9

NC, NS = 2, 16
NW = NC * NS
E_PER_W = E // NW
CHUNK = 80
NCHUNK = E_PER_W // CHUNK
NBUF = 3
PASS0 = 64
N_PAD = 10112
ROWS_PER_S = N_PAD // NS


def _sc_scatter_rows(table, gidx2, sidx3, ew):
    mesh = plsc.VectorSubcoreMesh(core_axis_name="c", subcore_axis_name="s")

    @functools.partial(
        pl.kernel,
        mesh=mesh,
        out_type=jax.ShapeDtypeStruct((NC, N_PAD, H), jnp.float32),
        scratch_types=[
            pltpu.VMEM((PASS0 * CHUNK,), jnp.int32),
            pltpu.VMEM((PASS0, CHUNK), jnp.int32),
            pltpu.VMEM((NBUF, CHUNK), jnp.float32),
            pltpu.VMEM((NBUF, CHUNK, H), jnp.float32),
            pltpu.VMEM_SHARED((N_PAD, H), jnp.float32),
        ] + [pltpu.SemaphoreType.DMA] * (3 * NBUF),
    )
    def k(table_hbm, gidx_hbm, sidx_hbm, ew_hbm, out_hbm,
          gi_v, si_v, ws_v, rows_v, acc_sh, *sems):
        c = lax.axis_index("c")
        s = lax.axis_index("s")
        wid = c * NS + s
        ebase = wid * E_PER_W

        zero16 = jnp.zeros((16,), jnp.float32)

        def zrow(i, carry):
            for j in range(H // 16):
                rows_v[0, i, pl.ds(j * 16, 16)] = zero16
            return carry

        lax.fori_loop(0, CHUNK, zrow, 0)
        zoff = s * ROWS_PER_S
        for k2 in range(ROWS_PER_S // CHUNK):
            pltpu.sync_copy(rows_v.at[0],
                            acc_sh.at[pl.ds(zoff + k2 * CHUNK, CHUNK)])
        ztail = ROWS_PER_S % CHUNK
        if ztail:
            pltpu.sync_copy(
                rows_v.at[0, pl.ds(0, ztail)],
                acc_sh.at[pl.ds(zoff + ROWS_PER_S - ztail, ztail)])
        plsc.subcore_barrier()

        def g_start(base, li, b):
            pltpu.async_copy(table_hbm.at[gi_v.at[pl.ds(li * CHUNK, CHUNK)]],
                             rows_v.at[b], sems[b])
            pltpu.async_copy(
                ew_hbm.at[pl.ds(ebase + (base + li) * CHUNK, CHUNK)],
                ws_v.at[b], sems[NBUF + b])

        def g_wait(b):
            pltpu.make_async_copy(table_hbm.at[pl.ds(0, CHUNK)],
                                  rows_v.at[b], sems[b]).wait()
            pltpu.make_async_copy(ew_hbm.at[pl.ds(0, CHUNK)],
                                  ws_v.at[b], sems[NBUF + b]).wait()

        def s_start(li, b):
            pltpu.async_copy(rows_v.at[b], acc_sh.at[si_v.at[li]],
                             sems[2 * NBUF + b], add=True)

        def s_wait(b):
            pltpu.make_async_copy(rows_v.at[b], acc_sh.at[pl.ds(0, CHUNK)],
                                  sems[2 * NBUF + b]).wait()

        def process(base, n_p, li, b, swait_prev):
            g_wait(b)

            def group16(g, carry):
                wv = ws_v[b, pl.ds(g * 16, 16)]
                row0 = g * 16
                for r in range(16):
                    wgt = wv[r]
                    for j in range(H // 16):
                        sl = pl.ds(j * 16, 16)
                        rows_v[b, row0 + r, sl] = rows_v[b, row0 + r, sl] * wgt
                return carry

            lax.fori_loop(0, CHUNK // 16, group16, 0)
            pb = (b + 2) % NBUF
            if swait_prev:
                s_wait(pb)

            if not (isinstance(li, int) and li + 2 >= n_p):
                @pl.when(jnp.int32(li) + 2 < n_p)
                def _():
                    g_start(base, li + 2, pb)

            s_start(li, b)

        for base, n_p in ((0, PASS0), (PASS0, NCHUNK - PASS0)):
            pltpu.sync_copy(
                gidx_hbm.at[pl.ds(ebase + base * CHUNK, n_p * CHUNK)],
                gi_v.at[pl.ds(0, n_p * CHUNK)])
            pltpu.sync_copy(sidx_hbm.at[wid, pl.ds(base, n_p)],
                            si_v.at[pl.ds(0, n_p)])
            g_start(base, 0, base % NBUF)
            g_start(base, 1, (base + 1) % NBUF)
            for li in range(3):
                process(base, n_p, li, (base + li) % NBUF, li > 0)
            ngrp = (n_p - 3) // NBUF

            def grp(i, carry):
                l0 = 3 + i * NBUF
                for k2 in range(NBUF):
                    process(base, n_p, l0 + k2, (base + 3 + k2) % NBUF, True)
                return carry

            lax.fori_loop(0, ngrp, grp, 0)
            for li in range(3 + ngrp * NBUF, n_p):
                process(base, n_p, li, (base + li) % NBUF, True)
            s_wait((base + n_p - 1) % NBUF)

        plsc.subcore_barrier()
        pltpu.sync_copy(acc_sh.at[pl.ds(s * ROWS_PER_S, ROWS_PER_S)],
                        out_hbm.at[c, pl.ds(s * ROWS_PER_S, ROWS_PER_S)])

    return k(table, gidx2, sidx3, ew)


_BR = 1000


def _tc_input_proj(x2, w_ih, b2):
    def body(xr, wr, br, outr):
        outr[...] = lax.dot_general(
            xr[...], wr[...], (((1,), (1,)), ((), ())),
            preferred_element_type=jnp.float32) + br[...]

    rows = T * N
    return pl.pallas_call(
        body,
        grid=(rows // _BR,),
        in_specs=[
            pl.BlockSpec((_BR, F), lambda i: (i, 0)),
            pl.BlockSpec((H, F), lambda i: (0, 0)),
            pl.BlockSpec((1, H), lambda i: (0, 0)),
        ],
        out_specs=pl.BlockSpec((_BR, H), lambda i: (i, 0)),
        out_shape=jax.ShapeDtypeStruct((rows, H), jnp.float32),
    )(x2, w_ih, b2)


def _tc_invdeg(d0, d1):
    def body(a, b, o):
        o[...] = 1.0 / (a[...] + b[...] + 1.0)

    return pl.pallas_call(
        body,
        grid=(N // _BR,),
        in_specs=[pl.BlockSpec((_BR, H), lambda i: (i, 0))] * 2,
        out_specs=pl.BlockSpec((_BR, H), lambda i: (i, 0)),
        out_shape=jax.ShapeDtypeStruct((N, H), jnp.float32),
    )(d0, d1)


def _tc_update(h, hd, m0, m1, ip_t, invdeg, w_hh):
    def body(hr, hdr, m0r, m1r, ipr, idr, wr, hor, hdor):
        z = m0r[...] + m1r[...] + hdr[...]
        u = lax.dot_general(z, wr[...], (((1,), (1,)), ((), ())),
                            preferred_element_type=jnp.float32)
        hn = (1.0 - ALPHA) * hr[...] + ALPHA * jnp.tanh(ipr[...] + u)
        hor[...] = hn
        hdor[...] = hn * idr[...]

    blk = pl.BlockSpec((_BR, H), lambda i: (i, 0))
    return pl.pallas_call(
        body,
        grid=(N // _BR,),
        in_specs=[blk, blk, blk, blk, blk,
                  blk, pl.BlockSpec((H, H), lambda i: (0, 0))],
        out_specs=[blk, blk],
        out_shape=[jax.ShapeDtypeStruct((N, H), jnp.float32)] * 2,
    )(h, hd, m0, m1, ip_t, invdeg, w_hh)


def kernel(x, edge_index, edge_weight, w_ih, w_hh, b_ih):
    src = edge_index[0]
    dst = edge_index[1]

    ssrc3 = src.reshape(NW, NCHUNK, CHUNK)
    sdst3 = dst.reshape(NW, NCHUNK, CHUNK)
    ew = edge_weight

    ones = jnp.ones((N, H), jnp.float32)
    degp = _sc_scatter_rows(ones, src, ssrc3, ew)
    invdeg = _tc_invdeg(degp[0, :N], degp[1, :N])

    ip = _tc_input_proj(x.reshape(T * N, F), w_ih, b_ih.reshape(1, H))
    ip = ip.reshape(T, N, H)

    zero = jnp.zeros((N, H), jnp.float32)
    h, hd, m0, m1 = zero, zero, zero, zero
    outs = []
    for t in range(T):
        h, hd = _tc_update(h, hd, m0, m1, ip[t], invdeg, w_hh)
        outs.append(h)
        if t + 1 < T:
            mp = _sc_scatter_rows(hd, src, sdst3, ew)
            m0, m1 = mp[0, :N], mp[1, :N]

    return jnp.stack(outs)[:, :, None, :]


# speedup vs baseline: 15.2314x; 1.0017x over previous
import functools

import jax
import jax.numpy as jnp
from jax import lax
from jax.experimental import pallas as pl
from jax.experimental.pallas import tpu as pltpu
from jax.experimental.pallas import tpu_sc as plsc

T, N, F, H, E = 12, 10000, 128, 128, 320000
ALPHA = 0.9

NC, NS = 2, 16
NW = NC * NS
E_PER_W = E // NW
CHUNK = 80
NCHUNK = E_PER_W // CHUNK
NBUF = 3
PASS0 = 64
N_PAD = 10112
ROWS_PER_S = N_PAD // NS


def _sc_scatter_rows(table, gidx2, sidx3, ew):
    mesh = plsc.VectorSubcoreMesh(core_axis_name="c", subcore_axis_name="s")

    @functools.partial(
        pl.kernel,
        mesh=mesh,
        out_type=jax.ShapeDtypeStruct((NC, N_PAD, H), jnp.float32),
        scratch_types=[
            pltpu.VMEM((PASS0 * CHUNK,), jnp.int32),
            pltpu.VMEM((PASS0, CHUNK), jnp.int32),
            pltpu.VMEM((NBUF, CHUNK), jnp.float32),
            pltpu.VMEM((NBUF, CHUNK, H), jnp.float32),
            pltpu.VMEM_SHARED((N_PAD, H), jnp.float32),
        ] + [pltpu.SemaphoreType.DMA] * (3 * NBUF),
    )
    def k(table_hbm, gidx_hbm, sidx_hbm, ew_hbm, out_hbm,
          gi_v, si_v, ws_v, rows_v, acc_sh, *sems):
        c = lax.axis_index("c")
        s = lax.axis_index("s")
        wid = c * NS + s
        ebase = wid * E_PER_W

        zero16 = jnp.zeros((16,), jnp.float32)

        def zrow(i, carry):
            for j in range(H // 16):
                rows_v[0, i, pl.ds(j * 16, 16)] = zero16
            return carry

        lax.fori_loop(0, CHUNK, zrow, 0)
        zoff = s * ROWS_PER_S
        for k2 in range(ROWS_PER_S // CHUNK):
            pltpu.sync_copy(rows_v.at[0],
                            acc_sh.at[pl.ds(zoff + k2 * CHUNK, CHUNK)])
        ztail = ROWS_PER_S % CHUNK
        if ztail:
            pltpu.sync_copy(
                rows_v.at[0, pl.ds(0, ztail)],
                acc_sh.at[pl.ds(zoff + ROWS_PER_S - ztail, ztail)])
        plsc.subcore_barrier()

        def g_start(base, li, b):
            pltpu.async_copy(table_hbm.at[gi_v.at[pl.ds(li * CHUNK, CHUNK)]],
                             rows_v.at[b], sems[b])
            pltpu.async_copy(
                ew_hbm.at[pl.ds(ebase + (base + li) * CHUNK, CHUNK)],
                ws_v.at[b], sems[NBUF + b])

        def g_wait(b):
            pltpu.make_async_copy(table_hbm.at[pl.ds(0, CHUNK)],
                                  rows_v.at[b], sems[b]).wait()
            pltpu.make_async_copy(ew_hbm.at[pl.ds(0, CHUNK)],
                                  ws_v.at[b], sems[NBUF + b]).wait()

        def s_start(li, b):
            pltpu.async_copy(rows_v.at[b], acc_sh.at[si_v.at[li]],
                             sems[2 * NBUF + b], add=True)

        def s_wait(b):
            pltpu.make_async_copy(rows_v.at[b], acc_sh.at[pl.ds(0, CHUNK)],
                                  sems[2 * NBUF + b]).wait()

        def process(base, n_p, li, b, swait_prev):
            g_wait(b)

            def group16(g, carry):
                wv = ws_v[b, pl.ds(g * 16, 16)]
                row0 = g * 16
                for r in range(16):
                    wgt = wv[r]
                    for j in range(H // 16):
                        sl = pl.ds(j * 16, 16)
                        rows_v[b, row0 + r, sl] = rows_v[b, row0 + r, sl] * wgt
                return carry

            lax.fori_loop(0, CHUNK // 16, group16, 0)
            s_start(li, b)
            pb = (b + 2) % NBUF
            if swait_prev:
                s_wait(pb)

            if not (isinstance(li, int) and li + 2 >= n_p):
                @pl.when(jnp.int32(li) + 2 < n_p)
                def _():
                    g_start(base, li + 2, pb)

        for base, n_p in ((0, PASS0), (PASS0, NCHUNK - PASS0)):
            pltpu.sync_copy(
                gidx_hbm.at[pl.ds(ebase + base * CHUNK, n_p * CHUNK)],
                gi_v.at[pl.ds(0, n_p * CHUNK)])
            pltpu.sync_copy(sidx_hbm.at[wid, pl.ds(base, n_p)],
                            si_v.at[pl.ds(0, n_p)])
            g_start(base, 0, base % NBUF)
            g_start(base, 1, (base + 1) % NBUF)
            for li in range(3):
                process(base, n_p, li, (base + li) % NBUF, li > 0)
            ngrp = (n_p - 3) // NBUF

            def grp(i, carry):
                l0 = 3 + i * NBUF
                for k2 in range(NBUF):
                    process(base, n_p, l0 + k2, (base + 3 + k2) % NBUF, True)
                return carry

            lax.fori_loop(0, ngrp, grp, 0)
            for li in range(3 + ngrp * NBUF, n_p):
                process(base, n_p, li, (base + li) % NBUF, True)
            s_wait((base + n_p - 1) % NBUF)

        plsc.subcore_barrier()
        pltpu.sync_copy(acc_sh.at[pl.ds(s * ROWS_PER_S, ROWS_PER_S)],
                        out_hbm.at[c, pl.ds(s * ROWS_PER_S, ROWS_PER_S)])

    return k(table, gidx2, sidx3, ew)


_BR = 1000


def _tc_input_proj(x2, w_ih, b2):
    def body(xr, wr, br, outr):
        outr[...] = lax.dot_general(
            xr[...], wr[...], (((1,), (1,)), ((), ())),
            preferred_element_type=jnp.float32) + br[...]

    rows = T * N
    return pl.pallas_call(
        body,
        grid=(rows // _BR,),
        in_specs=[
            pl.BlockSpec((_BR, F), lambda i: (i, 0)),
            pl.BlockSpec((H, F), lambda i: (0, 0)),
            pl.BlockSpec((1, H), lambda i: (0, 0)),
        ],
        out_specs=pl.BlockSpec((_BR, H), lambda i: (i, 0)),
        out_shape=jax.ShapeDtypeStruct((rows, H), jnp.float32),
    )(x2, w_ih, b2)


def _tc_invdeg(d0, d1):
    def body(a, b, o):
        o[...] = 1.0 / (a[...] + b[...] + 1.0)

    return pl.pallas_call(
        body,
        grid=(N // _BR,),
        in_specs=[pl.BlockSpec((_BR, H), lambda i: (i, 0))] * 2,
        out_specs=pl.BlockSpec((_BR, H), lambda i: (i, 0)),
        out_shape=jax.ShapeDtypeStruct((N, H), jnp.float32),
    )(d0, d1)


def _tc_update(h, hd, m0, m1, ip_t, invdeg, w_hh):
    def body(hr, hdr, m0r, m1r, ipr, idr, wr, hor, hdor):
        z = m0r[...] + m1r[...] + hdr[...]
        u = lax.dot_general(z, wr[...], (((1,), (1,)), ((), ())),
                            preferred_element_type=jnp.float32)
        hn = (1.0 - ALPHA) * hr[...] + ALPHA * jnp.tanh(ipr[...] + u)
        hor[...] = hn
        hdor[...] = hn * idr[...]

    blk = pl.BlockSpec((_BR, H), lambda i: (i, 0))
    return pl.pallas_call(
        body,
        grid=(N // _BR,),
        in_specs=[blk, blk, blk, blk, blk,
                  blk, pl.BlockSpec((H, H), lambda i: (0, 0))],
        out_specs=[blk, blk],
        out_shape=[jax.ShapeDtypeStruct((N, H), jnp.float32)] * 2,
    )(h, hd, m0, m1, ip_t, invdeg, w_hh)


def kernel(x, edge_index, edge_weight, w_ih, w_hh, b_ih):
    src = edge_index[0]
    dst = edge_index[1]

    ssrc3 = src.reshape(NW, NCHUNK, CHUNK)
    sdst3 = dst.reshape(NW, NCHUNK, CHUNK)
    ew = edge_weight

    ones = jnp.ones((N, H), jnp.float32)
    degp = _sc_scatter_rows(ones, src, ssrc3, ew)
    invdeg = _tc_invdeg(degp[0, :N], degp[1, :N])

    ip = _tc_input_proj(x.reshape(T * N, F), w_ih, b_ih.reshape(1, H))
    ip = ip.reshape(T, N, H)

    zero = jnp.zeros((N, H), jnp.float32)
    h, hd, m0, m1 = zero, zero, zero, zero
    outs = []
    for t in range(T):
        h, hd = _tc_update(h, hd, m0, m1, ip[t], invdeg, w_hh)
        outs.append(h)
        if t + 1 < T:
            mp = _sc_scatter_rows(hd, src, sdst3, ew)
            m0, m1 = mp[0, :N], mp[1, :N]

    return jnp.stack(outs)[:, :, None, :]
